# baseline (speedup 1.0000x reference)
import functools

import jax
import jax.numpy as jnp
from jax import lax
from jax.experimental import pallas as pl
from jax.experimental.pallas import tpu as pltpu
from jax.experimental.pallas import tpu_sc as plsc

_B = 16384
_V = 100000
_D = 4
_F_NS = 26
_F_SEQ = 4
_T = 50

_NW = 32
_BW = _B // _NW
_CB = 32
_NCHUNK = _BW // _CB

_NS_ROWS = _CB * _F_NS
_NS_ELEMS = _NS_ROWS * _D
_SEQ_ROWS = _CB * _F_SEQ * _T
_OUT_SEQ = _CB * _F_SEQ * _D


def _body(ns_t0, ns_t1, ns_t2, ns_t3, seq_t0, seq_t1, seq_t2, seq_t3,
          ns_idx, seq_idx, fo_seq, fo_ns, out_ns, out_seq,
          raw_seq0, raw_seq1, idx_seq0, idx_seq1, seq_vals0, seq_vals1,
          raw_ns0, raw_ns1, idx_ns0, idx_ns1, ns_vals0, ns_vals1,
          fo_seq_v, fo_ns_v, ns_out_v, stage_v,
          sem_seq0, sem_seq1, sem_ns0, sem_ns1):
    ns_tabs = (ns_t0, ns_t1, ns_t2, ns_t3)
    seq_tabs = (seq_t0, seq_t1, seq_t2, seq_t3)
    raw_seq = (raw_seq0, raw_seq1)
    idx_seq = (idx_seq0, idx_seq1)
    seq_vals = (seq_vals0, seq_vals1)
    raw_ns = (raw_ns0, raw_ns1)
    idx_ns = (idx_ns0, idx_ns1)
    ns_vals = (ns_vals0, ns_vals1)
    sem_seq = (sem_seq0, sem_seq1)
    sem_ns = (sem_ns0, sem_ns1)

    wid = lax.axis_index("s") * 2 + lax.axis_index("c")

    iota = lax.iota(jnp.int32, 16)
    rowq = iota >> 2
    colr = iota & 3

    pltpu.sync_copy(fo_seq, fo_seq_v)
    pltpu.sync_copy(fo_ns, fo_ns_v)

    def stage_and_fire(c, slot):
        blk = wid * _NCHUNK + c
        pltpu.sync_copy(seq_idx.at[pl.ds(blk * _SEQ_ROWS, _SEQ_ROWS)],
                        raw_seq[slot])
        def _exp_seq(s, carry):
            idx_seq[slot][pl.ds(s * 16, 16)] = (
                raw_seq[slot][pl.ds(s * 16, 16)]
                + fo_seq_v[pl.ds(s * 16, 16)])
            return carry
        lax.fori_loop(0, _SEQ_ROWS // 16, _exp_seq, 0)
        seq_cps = [
            pltpu.async_copy(
                seq_tabs[d].at[idx_seq[slot]],
                seq_vals[slot].at[pl.ds(d * _SEQ_ROWS, _SEQ_ROWS)],
                sem_seq[slot])
            for d in range(_D)
        ]

        pltpu.sync_copy(ns_idx.at[pl.ds(blk * _NS_ROWS, _NS_ROWS)],
                        raw_ns[slot])
        def _exp_ns(s, carry):
            idx_ns[slot][pl.ds(s * 16, 16)] = (
                raw_ns[slot][pl.ds(s * 16, 16)]
                + fo_ns_v[pl.ds(s * 16, 16)])
            return carry
        lax.fori_loop(0, _NS_ROWS // 16, _exp_ns, 0)
        ns_cps = [
            pltpu.async_copy(
                ns_tabs[d].at[idx_ns[slot]],
                ns_vals[slot].at[pl.ds(d * _NS_ROWS, _NS_ROWS)],
                sem_ns[slot])
            for d in range(_D)
        ]
        return seq_cps, ns_cps

    def drain(c, slot, cps):
        blk = wid * _NCHUNK + c
        seq_cps, ns_cps = cps
        for cp in ns_cps:
            cp.wait()
        def _il_ns(s, carry):
            pos = (s * 16 + iota) * 4
            for d in range(_D):
                v = ns_vals[slot][pl.ds(d * _NS_ROWS + s * 16, 16)]
                plsc.store_scatter(ns_out_v, [pos + d], v)
            return carry
        lax.fori_loop(0, _NS_ROWS // 16, _il_ns, 0)
        pltpu.sync_copy(ns_out_v,
                        out_ns.at[pl.ds(blk * _NS_ELEMS, _NS_ELEMS)])

        for cp in seq_cps:
            cp.wait()
        def _pool(k, carry):
            base = colr * _SEQ_ROWS + 200 * k + _T * rowq
            def _t(t, accs):
                a0, a1 = accs
                a0 = a0 + plsc.load_gather(seq_vals[slot], [base + t])
                a1 = a1 + plsc.load_gather(seq_vals[slot], [base + t + 25])
                return a0, a1
            a0, a1 = lax.fori_loop(0, _T // 2, _t,
                                   (jnp.zeros((16,), jnp.float32),
                                    jnp.zeros((16,), jnp.float32)))
            stage_v[pl.ds(k * 16, 16)] = (a0 + a1) * (1.0 / _T)
            return carry
        lax.fori_loop(0, _OUT_SEQ // 16, _pool, 0)
        pltpu.sync_copy(stage_v,
                        out_seq.at[pl.ds(blk * _OUT_SEQ, _OUT_SEQ)])

    cps = stage_and_fire(0, 0)
    for c in range(_NCHUNK):
        slot = c & 1
        if c + 1 < _NCHUNK:
            next_cps = stage_and_fire(c + 1, 1 - slot)
        drain(c, slot, cps)
        if c + 1 < _NCHUNK:
            cps = next_cps


@jax.jit
def _sc_call(ns_tabs, seq_tabs, ns_idx, seq_idx, fo_seq, fo_ns):
    mesh = plsc.VectorSubcoreMesh(
        core_axis_name="c", subcore_axis_name="s",
        num_cores=2, num_subcores=16)
    f = functools.partial(
        pl.kernel,
        out_type=(
            jax.ShapeDtypeStruct((_B * _F_NS * _D,), jnp.float32),
            jax.ShapeDtypeStruct((_B * _F_SEQ * _D,), jnp.float32),
        ),
        mesh=mesh,
        compiler_params=pltpu.CompilerParams(
            needs_layout_passes=False,
            use_tc_tiling_on_sc=False,
        ),
        scratch_types=(
            [pltpu.VMEM((_SEQ_ROWS,), jnp.int32)] * 4
            + [pltpu.VMEM((_SEQ_ROWS * _D,), jnp.float32)] * 2
            + [pltpu.VMEM((_NS_ROWS,), jnp.int32)] * 4
            + [pltpu.VMEM((_NS_ELEMS,), jnp.float32)] * 2
            + [pltpu.VMEM((_SEQ_ROWS,), jnp.int32),
               pltpu.VMEM((_NS_ROWS,), jnp.int32),
               pltpu.VMEM((_NS_ELEMS,), jnp.float32),
               pltpu.VMEM((_OUT_SEQ,), jnp.float32)]
            + [pltpu.SemaphoreType.DMA] * 4
        ),
    )(_body)
    return f(*ns_tabs, *seq_tabs, ns_idx, seq_idx, fo_seq, fo_ns)


def kernel(ns_numeric, ns_sparse_idx, seq_sparse_idx, ns_tables, seq_tables):
    b = ns_sparse_idx.shape[0]
    j_seq = jnp.arange(_SEQ_ROWS, dtype=jnp.int32)
    fo_seq = ((j_seq // _T) % _F_SEQ) * _V
    j_ns = jnp.arange(_NS_ROWS, dtype=jnp.int32)
    fo_ns = (j_ns % _F_NS) * _V

    ns_tabs = tuple(ns_tables[:, :, d].reshape(-1) for d in range(_D))
    seq_tabs = tuple(seq_tables[:, :, d].reshape(-1) for d in range(_D))

    out_ns, out_seq = _sc_call(
        ns_tabs, seq_tabs,
        ns_sparse_idx.reshape(-1), seq_sparse_idx.reshape(-1),
        fo_seq, fo_ns)

    return jnp.concatenate(
        [out_ns.reshape(b, _F_NS * _D), ns_numeric,
         out_seq.reshape(b, _F_SEQ * _D)], axis=1)


# speedup vs baseline: 1.0705x; 1.0705x over previous
import functools

import jax
import jax.numpy as jnp
from jax import lax
from jax.experimental import pallas as pl
from jax.experimental.pallas import tpu as pltpu
from jax.experimental.pallas import tpu_sc as plsc

_B = 16384
_V = 100000
_D = 4
_F_NS = 26
_F_SEQ = 4
_T = 50

_NW = 32
_BW = _B // _NW

_CBS = 32
_NCH_S = _BW // _CBS
_SEQ_ROWS = _CBS * _F_SEQ * _T
_OUT_SEQ = _CBS * _F_SEQ * _D

_CBN = 64
_NCH_N = _BW // _CBN
_NS_ROWS = _CBN * _F_NS
_NS_ELEMS = _NS_ROWS * _D

_MESH = dict(core_axis_name="c", subcore_axis_name="s",
             num_cores=2, num_subcores=16)
_CPARAMS = pltpu.CompilerParams(
    needs_layout_passes=False, use_tc_tiling_on_sc=False)


def _seq_body(t0, t1, t2, t3, seq_idx, fo_seq, out_seq,
              raw0, raw1, idx0, idx1, vals0, vals1, fo_v, stage_v,
              sem0, sem1):
    tabs = (t0, t1, t2, t3)
    raw = (raw0, raw1)
    idx = (idx0, idx1)
    vals = (vals0, vals1)
    sems = (sem0, sem1)
    wid = lax.axis_index("s") * 2 + lax.axis_index("c")

    iota = lax.iota(jnp.int32, 16)
    rowq = iota >> 2
    colr = iota & 3

    pltpu.sync_copy(fo_seq, fo_v)

    def stage_and_fire(c, slot):
        blk = wid * _NCH_S + c
        pltpu.sync_copy(seq_idx.at[pl.ds(blk * _SEQ_ROWS, _SEQ_ROWS)],
                        raw[slot])
        def _exp(s, carry):
            idx[slot][pl.ds(s * 16, 16)] = (raw[slot][pl.ds(s * 16, 16)]
                                            + fo_v[pl.ds(s * 16, 16)])
            return carry
        lax.fori_loop(0, _SEQ_ROWS // 16, _exp, 0)
        return [
            pltpu.async_copy(
                tabs[d].at[idx[slot]],
                vals[slot].at[pl.ds(d * _SEQ_ROWS, _SEQ_ROWS)],
                sems[slot])
            for d in range(_D)
        ]

    def drain(c, slot, cps):
        blk = wid * _NCH_S + c
        for cp in cps:
            cp.wait()
        def _pool(k, carry):
            base = colr * _SEQ_ROWS + 200 * k + _T * rowq
            def _t(t, accs):
                a0, a1 = accs
                a0 = a0 + plsc.load_gather(vals[slot], [base + t])
                a1 = a1 + plsc.load_gather(vals[slot], [base + t + 25])
                return a0, a1
            a0, a1 = lax.fori_loop(0, _T // 2, _t,
                                   (jnp.zeros((16,), jnp.float32),
                                    jnp.zeros((16,), jnp.float32)))
            stage_v[pl.ds(k * 16, 16)] = (a0 + a1) * (1.0 / _T)
            return carry
        lax.fori_loop(0, _OUT_SEQ // 16, _pool, 0)
        pltpu.sync_copy(stage_v, out_seq.at[pl.ds(blk * _OUT_SEQ, _OUT_SEQ)])

    cps = stage_and_fire(0, 0)
    for c in range(_NCH_S):
        if c + 1 < _NCH_S:
            nxt = stage_and_fire(c + 1, (c + 1) & 1)
        drain(c, c & 1, cps)
        if c + 1 < _NCH_S:
            cps = nxt


def _ns_body(t0, t1, t2, t3, ns_idx, fo_ns, out_ns,
             raw0, raw1, idx0, idx1, vals0, vals1, fo_v, out_v,
             sem0, sem1):
    tabs = (t0, t1, t2, t3)
    raw = (raw0, raw1)
    idx = (idx0, idx1)
    vals = (vals0, vals1)
    sems = (sem0, sem1)
    wid = lax.axis_index("s") * 2 + lax.axis_index("c")

    iota = lax.iota(jnp.int32, 16)

    pltpu.sync_copy(fo_ns, fo_v)

    def stage_and_fire(c, slot):
        blk = wid * _NCH_N + c
        pltpu.sync_copy(ns_idx.at[pl.ds(blk * _NS_ROWS, _NS_ROWS)],
                        raw[slot])
        def _exp(s, carry):
            idx[slot][pl.ds(s * 16, 16)] = (raw[slot][pl.ds(s * 16, 16)]
                                            + fo_v[pl.ds(s * 16, 16)])
            return carry
        lax.fori_loop(0, _NS_ROWS // 16, _exp, 0)
        return [
            pltpu.async_copy(
                tabs[d].at[idx[slot]],
                vals[slot].at[pl.ds(d * _NS_ROWS, _NS_ROWS)],
                sems[slot])
            for d in range(_D)
        ]

    def drain(c, slot, cps):
        blk = wid * _NCH_N + c
        for cp in cps:
            cp.wait()
        def _il(s, carry):
            pos = (s * 16 + iota) * 4
            for d in range(_D):
                v = vals[slot][pl.ds(d * _NS_ROWS + s * 16, 16)]
                plsc.store_scatter(out_v, [pos + d], v)
            return carry
        lax.fori_loop(0, _NS_ROWS // 16, _il, 0)
        pltpu.sync_copy(out_v, out_ns.at[pl.ds(blk * _NS_ELEMS, _NS_ELEMS)])

    cps = stage_and_fire(0, 0)
    for c in range(_NCH_N):
        if c + 1 < _NCH_N:
            nxt = stage_and_fire(c + 1, (c + 1) & 1)
        drain(c, c & 1, cps)
        if c + 1 < _NCH_N:
            cps = nxt


@jax.jit
def _sc_calls(ns_tabs, seq_tabs, ns_idx, seq_idx, fo_seq, fo_ns):
    seq_f = functools.partial(
        pl.kernel,
        out_type=jax.ShapeDtypeStruct((_B * _F_SEQ * _D,), jnp.float32),
        mesh=plsc.VectorSubcoreMesh(**_MESH),
        compiler_params=_CPARAMS,
        scratch_types=(
            [pltpu.VMEM((_SEQ_ROWS,), jnp.int32)] * 4
            + [pltpu.VMEM((_SEQ_ROWS * _D,), jnp.float32)] * 2
            + [pltpu.VMEM((_SEQ_ROWS,), jnp.int32),
               pltpu.VMEM((_OUT_SEQ,), jnp.float32)]
            + [pltpu.SemaphoreType.DMA] * 2
        ),
    )(_seq_body)
    out_seq = seq_f(*seq_tabs, seq_idx, fo_seq)

    ns_f = functools.partial(
        pl.kernel,
        out_type=jax.ShapeDtypeStruct((_B * _F_NS * _D,), jnp.float32),
        mesh=plsc.VectorSubcoreMesh(**_MESH),
        compiler_params=_CPARAMS,
        scratch_types=(
            [pltpu.VMEM((_NS_ROWS,), jnp.int32)] * 4
            + [pltpu.VMEM((_NS_ELEMS,), jnp.float32)] * 2
            + [pltpu.VMEM((_NS_ROWS,), jnp.int32),
               pltpu.VMEM((_NS_ELEMS,), jnp.float32)]
            + [pltpu.SemaphoreType.DMA] * 2
        ),
    )(_ns_body)
    out_ns = ns_f(*ns_tabs, ns_idx, fo_ns)
    return out_ns, out_seq


def kernel(ns_numeric, ns_sparse_idx, seq_sparse_idx, ns_tables, seq_tables):
    b = ns_sparse_idx.shape[0]
    j_seq = jnp.arange(_SEQ_ROWS, dtype=jnp.int32)
    fo_seq = ((j_seq // _T) % _F_SEQ) * _V
    j_ns = jnp.arange(_NS_ROWS, dtype=jnp.int32)
    fo_ns = (j_ns % _F_NS) * _V

    ns_tabs = tuple(ns_tables[:, :, d].reshape(-1) for d in range(_D))
    seq_tabs = tuple(seq_tables[:, :, d].reshape(-1) for d in range(_D))

    out_ns, out_seq = _sc_calls(
        ns_tabs, seq_tabs,
        ns_sparse_idx.reshape(-1), seq_sparse_idx.reshape(-1),
        fo_seq, fo_ns)

    return jnp.concatenate(
        [out_ns.reshape(b, _F_NS * _D), ns_numeric,
         out_seq.reshape(b, _F_SEQ * _D)], axis=1)


# speedup vs baseline: 1.2831x; 1.1985x over previous
import functools

import jax
import jax.numpy as jnp
from jax import lax
from jax.experimental import pallas as pl
from jax.experimental.pallas import tpu as pltpu
from jax.experimental.pallas import tpu_sc as plsc

_B = 16384
_V = 100000
_D = 4
_F_NS = 26
_F_SEQ = 4
_T = 50

_NW = 32
_BW = _B // _NW

_CBS = 32
_NCH_S = _BW // _CBS
_SEQ_ROWS = _CBS * _F_SEQ * _T
_OUT_SEQ = _CBS * _F_SEQ * _D

_CBN = 64
_NCH_N = _BW // _CBN
_NS_ROWS = _CBN * _F_NS
_NS_ELEMS = _NS_ROWS * _D

_MESH = dict(core_axis_name="c", subcore_axis_name="s",
             num_cores=2, num_subcores=16)
_CPARAMS = pltpu.CompilerParams(
    needs_layout_passes=False, use_tc_tiling_on_sc=False)


def _seq_body(t0, t1, t2, t3, seq_idx, fo_seq, out_seq,
              raw0, raw1, idx0, idx1, vals0, vals1, fo_v, stage_v,
              sem0, sem1):
    tabs = (t0, t1, t2, t3)
    raw = (raw0, raw1)
    idx = (idx0, idx1)
    vals = (vals0, vals1)
    sems = (sem0, sem1)
    wid = lax.axis_index("s") * 2 + lax.axis_index("c")

    iota = lax.iota(jnp.int32, 16)
    rowq = iota >> 2
    colr = iota & 3

    pltpu.sync_copy(fo_seq, fo_v)

    def stage_and_fire(c, slot):
        blk = wid * _NCH_S + c
        pltpu.sync_copy(seq_idx.at[pl.ds(blk * _SEQ_ROWS, _SEQ_ROWS)],
                        raw[slot])
        def _exp(s, carry):
            idx[slot][pl.ds(s * 16, 16)] = (raw[slot][pl.ds(s * 16, 16)]
                                            + fo_v[pl.ds(s * 16, 16)])
            return carry
        lax.fori_loop(0, _SEQ_ROWS // 16, _exp, 0)
        return [
            pltpu.async_copy(
                tabs[d].at[idx[slot]],
                vals[slot].at[pl.ds(d * _SEQ_ROWS, _SEQ_ROWS)],
                sems[slot])
            for d in range(_D)
        ]

    def drain(c, slot, cps):
        blk = wid * _NCH_S + c
        for cp in cps:
            cp.wait()
        def _pool(k, carry):
            base = colr * _SEQ_ROWS + 200 * k + _T * rowq
            def _t(t, accs):
                a0, a1 = accs
                a0 = a0 + plsc.load_gather(vals[slot], [base + t])
                a1 = a1 + plsc.load_gather(vals[slot], [base + t + 25])
                return a0, a1
            a0, a1 = lax.fori_loop(0, _T // 2, _t,
                                   (jnp.zeros((16,), jnp.float32),
                                    jnp.zeros((16,), jnp.float32)))
            stage_v[pl.ds(k * 16, 16)] = (a0 + a1) * (1.0 / _T)
            return carry
        lax.fori_loop(0, _OUT_SEQ // 16, _pool, 0)
        pltpu.sync_copy(stage_v, out_seq.at[pl.ds(blk * _OUT_SEQ, _OUT_SEQ)])

    cps = stage_and_fire(0, 0)
    for c in range(_NCH_S):
        if c + 1 < _NCH_S:
            nxt = stage_and_fire(c + 1, (c + 1) & 1)
        drain(c, c & 1, cps)
        if c + 1 < _NCH_S:
            cps = nxt


def _ns_body(t0, t1, t2, t3, ns_idx, fo_ns, tok, out_ns,
             raw0, raw1, idx0, idx1, vals0, vals1, fo_v, out_v,
             sem0, sem1):
    tabs = (t0, t1, t2, t3)
    raw = (raw0, raw1)
    idx = (idx0, idx1)
    vals = (vals0, vals1)
    sems = (sem0, sem1)
    wid = lax.axis_index("s") * 2 + lax.axis_index("c")

    iota = lax.iota(jnp.int32, 16)

    pltpu.sync_copy(fo_ns, fo_v)

    def stage_and_fire(c, slot):
        blk = wid * _NCH_N + c
        pltpu.sync_copy(ns_idx.at[pl.ds(blk * _NS_ROWS, _NS_ROWS)],
                        raw[slot])
        def _exp(s, carry):
            idx[slot][pl.ds(s * 16, 16)] = (raw[slot][pl.ds(s * 16, 16)]
                                            + fo_v[pl.ds(s * 16, 16)])
            return carry
        lax.fori_loop(0, _NS_ROWS // 16, _exp, 0)
        return [
            pltpu.async_copy(
                tabs[d].at[idx[slot]],
                vals[slot].at[pl.ds(d * _NS_ROWS, _NS_ROWS)],
                sems[slot])
            for d in range(_D)
        ]

    def drain(c, slot, cps):
        blk = wid * _NCH_N + c
        for cp in cps:
            cp.wait()
        def _il(s, carry):
            pos = (s * 16 + iota) * 4
            for d in range(_D):
                v = vals[slot][pl.ds(d * _NS_ROWS + s * 16, 16)]
                plsc.store_scatter(out_v, [pos + d], v)
            return carry
        lax.fori_loop(0, _NS_ROWS // 16, _il, 0)
        pltpu.sync_copy(out_v, out_ns.at[pl.ds(blk * _NS_ELEMS, _NS_ELEMS)])

    cps = stage_and_fire(0, 0)
    for c in range(_NCH_N):
        if c + 1 < _NCH_N:
            nxt = stage_and_fire(c + 1, (c + 1) & 1)
        drain(c, c & 1, cps)
        if c + 1 < _NCH_N:
            cps = nxt


@jax.jit
def _sc_calls(ns_tabs, seq_tabs, ns_idx, seq_idx, fo_seq, fo_ns):
    seq_f = functools.partial(
        pl.kernel,
        out_type=jax.ShapeDtypeStruct((_B * _F_SEQ * _D,), jnp.float32),
        mesh=plsc.VectorSubcoreMesh(**_MESH),
        compiler_params=_CPARAMS,
        scratch_types=(
            [pltpu.VMEM((_SEQ_ROWS,), jnp.int32)] * 4
            + [pltpu.VMEM((_SEQ_ROWS * _D,), jnp.float32)] * 2
            + [pltpu.VMEM((_SEQ_ROWS,), jnp.int32),
               pltpu.VMEM((_OUT_SEQ,), jnp.float32)]
            + [pltpu.SemaphoreType.DMA] * 2
        ),
    )(_seq_body)
    out_seq = seq_f(*seq_tabs, seq_idx, fo_seq)

    ns_f = functools.partial(
        pl.kernel,
        out_type=jax.ShapeDtypeStruct((_B * _F_NS * _D,), jnp.float32),
        mesh=plsc.VectorSubcoreMesh(**_MESH),
        compiler_params=_CPARAMS,
        scratch_types=(
            [pltpu.VMEM((_NS_ROWS,), jnp.int32)] * 4
            + [pltpu.VMEM((_NS_ELEMS,), jnp.float32)] * 2
            + [pltpu.VMEM((_NS_ROWS,), jnp.int32),
               pltpu.VMEM((_NS_ELEMS,), jnp.float32)]
            + [pltpu.SemaphoreType.DMA] * 2
        ),
    )(_ns_body)
    tok = lax.slice(out_seq, (0,), (8,))
    out_ns = ns_f(*ns_tabs, ns_idx, fo_ns, tok)
    return out_ns, out_seq


def kernel(ns_numeric, ns_sparse_idx, seq_sparse_idx, ns_tables, seq_tables):
    b = ns_sparse_idx.shape[0]
    j_seq = jnp.arange(_SEQ_ROWS, dtype=jnp.int32)
    fo_seq = ((j_seq // _T) % _F_SEQ) * _V
    j_ns = jnp.arange(_NS_ROWS, dtype=jnp.int32)
    fo_ns = (j_ns % _F_NS) * _V

    ns_tabs = tuple(ns_tables[:, :, d].reshape(-1) for d in range(_D))
    seq_tabs = tuple(seq_tables[:, :, d].reshape(-1) for d in range(_D))

    out_ns, out_seq = _sc_calls(
        ns_tabs, seq_tabs,
        ns_sparse_idx.reshape(-1), seq_sparse_idx.reshape(-1),
        fo_seq, fo_ns)

    return jnp.concatenate(
        [out_ns.reshape(b, _F_NS * _D), ns_numeric,
         out_seq.reshape(b, _F_SEQ * _D)], axis=1)
